# baseline (device time: 36063 ns/iter reference)
import jax
import jax.numpy as jnp
from jax import lax
from jax.experimental import pallas as pl
from jax.experimental.pallas import tpu as pltpu

N_DEV = 4
B = 2
S = 512
H = 8
D = 64
HD = H * D
WIN = 128
E = 768
SKV = S + 2 * WIN


def kernel(x, Wq, K_ext, V_ext, Wo):
    K2 = K_ext.reshape(B, S, HD)
    V2 = V_ext.reshape(B, S, HD)

    def body(x_ref, wq_ref, k_ref, v_ref, wo_ref, out_ref,
             k_lo, k_hi, v_lo, v_hi, send_sems, recv_sems):
        my = lax.axis_index("i")
        left = lax.rem(my + N_DEV - 1, N_DEV)
        right = lax.rem(my + 1, N_DEV)

        barrier = pltpu.get_barrier_semaphore()
        for nbr in (left, right):
            pl.semaphore_signal(barrier, inc=1, device_id=(nbr,),
                                device_id_type=pl.DeviceIdType.MESH)
        pl.semaphore_wait(barrier, 2)

        rdmas = []
        for idx, (src, dst, nbr) in enumerate([
            (k_ref.at[:, pl.ds(0, WIN), :], k_hi, left),
            (v_ref.at[:, pl.ds(0, WIN), :], v_hi, left),
            (k_ref.at[:, pl.ds(S - WIN, WIN), :], k_lo, right),
            (v_ref.at[:, pl.ds(S - WIN, WIN), :], v_lo, right),
        ]):
            rdma = pltpu.make_async_remote_copy(
                src_ref=src, dst_ref=dst,
                send_sem=send_sems.at[idx], recv_sem=recv_sems.at[idx],
                device_id=(nbr,), device_id_type=pl.DeviceIdType.MESH,
            )
            rdma.start()
            rdmas.append(rdma)

        wq = wq_ref[...].astype(jnp.bfloat16)
        qs = []
        for b in range(B):
            xb = x_ref[b].astype(jnp.bfloat16)
            q = lax.dot(xb, wq, preferred_element_type=jnp.float32)
            qs.append(q.astype(jnp.bfloat16))

        for r in rdmas:
            r.wait()

        row = lax.broadcasted_iota(jnp.int32, (S, SKV), 0)
        col = lax.broadcasted_iota(jnp.int32, (S, SKV), 1)
        kj = my * S - WIN + col
        valid = ((col >= row) & (col <= row + 2 * WIN)
                 & (kj >= 0) & (kj < N_DEV * S))

        wo = wo_ref[...].astype(jnp.bfloat16)
        for b in range(B):
            kfull = jnp.concatenate(
                [k_lo[b], k_ref[b], k_hi[b]], axis=0).astype(jnp.bfloat16)
            vfull = jnp.concatenate(
                [v_lo[b], v_ref[b], v_hi[b]], axis=0).astype(jnp.bfloat16)
            ctxs = []
            for h in range(H):
                qh = qs[b][:, h * D:(h + 1) * D]
                kh = kfull[:, h * D:(h + 1) * D]
                scores = lax.dot_general(
                    qh, kh, (((1,), (1,)), ((), ())),
                    preferred_element_type=jnp.float32) * 0.125
                scores = jnp.where(valid, scores, jnp.float32(-1e9))
                m = jnp.max(scores, axis=1, keepdims=True)
                w = jnp.exp(scores - m)
                w = w / jnp.sum(w, axis=1, keepdims=True)
                vh = vfull[:, h * D:(h + 1) * D]
                ctxs.append(lax.dot(w.astype(jnp.bfloat16), vh,
                                    preferred_element_type=jnp.float32))
            ctx = jnp.concatenate(ctxs, axis=1).astype(jnp.bfloat16)
            out_ref[b] = lax.dot(ctx, wo, preferred_element_type=jnp.float32)

    return pl.pallas_call(
        body,
        out_shape=jax.ShapeDtypeStruct((B, S, E), jnp.float32),
        in_specs=[pl.BlockSpec(memory_space=pltpu.VMEM)] * 5,
        out_specs=pl.BlockSpec(memory_space=pltpu.VMEM),
        scratch_shapes=[
            pltpu.VMEM((B, WIN, HD), jnp.float32),
            pltpu.VMEM((B, WIN, HD), jnp.float32),
            pltpu.VMEM((B, WIN, HD), jnp.float32),
            pltpu.VMEM((B, WIN, HD), jnp.float32),
            pltpu.SemaphoreType.DMA((4,)),
            pltpu.SemaphoreType.DMA((4,)),
        ],
        compiler_params=pltpu.CompilerParams(collective_id=0),
    )(x, Wq, K2, V2, Wo)


# device time: 31727 ns/iter; 1.1367x vs baseline; 1.1367x over previous
import jax
import jax.numpy as jnp
from jax import lax
from jax.experimental import pallas as pl
from jax.experimental.pallas import tpu as pltpu

N_DEV = 4
B = 2
S = 512
H = 8
D = 64
HD = H * D
WIN = 128
E = 768
T = 4
BLK = S // T
W = BLK + 2 * WIN

BF = jnp.bfloat16


def kernel(x, Wq, K_ext, V_ext, Wo):
    xb = x.astype(BF)
    K2 = K_ext.reshape(B, S, HD).astype(BF)
    V2 = V_ext.reshape(B, S, HD).astype(BF)
    Wqb = Wq.astype(BF)
    Wob = Wo.astype(BF)

    def body(x_ref, wq_ref, k_ref, v_ref, wo_ref, out_ref,
             k_lo, k_hi, v_lo, v_hi, send_sems, recv_sems):
        my = lax.axis_index("i")
        left = lax.rem(my + N_DEV - 1, N_DEV)
        right = lax.rem(my + 1, N_DEV)

        barrier = pltpu.get_barrier_semaphore()
        for nbr in (left, right):
            pl.semaphore_signal(barrier, inc=1, device_id=(nbr,),
                                device_id_type=pl.DeviceIdType.MESH)
        pl.semaphore_wait(barrier, 2)

        rdmas = []
        for idx, (src, dst, nbr) in enumerate([
            (k_ref.at[:, pl.ds(0, WIN), :], k_hi, left),
            (v_ref.at[:, pl.ds(0, WIN), :], v_hi, left),
            (k_ref.at[:, pl.ds(S - WIN, WIN), :], k_lo, right),
            (v_ref.at[:, pl.ds(S - WIN, WIN), :], v_lo, right),
        ]):
            rdma = pltpu.make_async_remote_copy(
                src_ref=src, dst_ref=dst,
                send_sem=send_sems.at[idx], recv_sem=recv_sems.at[idx],
                device_id=(nbr,), device_id_type=pl.DeviceIdType.MESH,
            )
            rdma.start()
            rdmas.append(rdma)


        wq = wq_ref[...]
        qs = []
        for b in range(B):
            q = lax.dot(x_ref[b], wq, preferred_element_type=jnp.float32)
            qs.append((q * 0.125).astype(BF))

        li = lax.broadcasted_iota(jnp.int32, (BLK, W), 0)
        lj = lax.broadcasted_iota(jnp.int32, (BLK, W), 1)
        biases = []
        for t in range(T):
            kj = my * S - WIN + t * BLK + lj
            cond = ((lj >= li) & (lj <= li + 2 * WIN)
                    & (kj >= 0) & (kj < N_DEV * S))
            biases.append(jnp.where(cond, 0.0, -1e9).astype(jnp.float32))

        wo = wo_ref[...]
        ctx = [[None] * T for _ in range(B)]

        def attn_block(b, t, kwin, vwin):
            parts = []
            for h in range(H):
                qh = qs[b][t * BLK:(t + 1) * BLK, h * D:(h + 1) * D]
                s = lax.dot_general(
                    qh, kwin[:, h * D:(h + 1) * D],
                    (((1,), (1,)), ((), ())),
                    preferred_element_type=jnp.float32)
                w = jnp.exp(s + biases[t])
                w = w * (1.0 / jnp.sum(w, axis=1, keepdims=True))
                parts.append(lax.dot(w.astype(BF), vwin[:, h * D:(h + 1) * D],
                                     preferred_element_type=jnp.float32))
            ctx[b][t] = jnp.concatenate(parts, axis=1).astype(BF)

        for b in range(B):
            kb = k_ref[b]
            vb = v_ref[b]
            attn_block(b, 1, kb[0:W], vb[0:W])
            attn_block(b, 2, kb[S - W:S], vb[S - W:S])

        for r in rdmas:
            r.wait()

        for b in range(B):
            k0 = jnp.concatenate([k_lo[b], k_ref[b, 0:W - WIN]], axis=0)
            v0 = jnp.concatenate([v_lo[b], v_ref[b, 0:W - WIN]], axis=0)
            attn_block(b, 0, k0, v0)
            k3 = jnp.concatenate([k_ref[b, S - (W - WIN):S], k_hi[b]], axis=0)
            v3 = jnp.concatenate([v_ref[b, S - (W - WIN):S], v_hi[b]], axis=0)
            attn_block(b, 3, k3, v3)

        for b in range(B):
            for t in range(T):
                out_ref[b, t * BLK:(t + 1) * BLK, :] = lax.dot(
                    ctx[b][t], wo, preferred_element_type=jnp.float32)

    return pl.pallas_call(
        body,
        out_shape=jax.ShapeDtypeStruct((B, S, E), jnp.float32),
        in_specs=[pl.BlockSpec(memory_space=pltpu.VMEM)] * 5,
        out_specs=pl.BlockSpec(memory_space=pltpu.VMEM),
        scratch_shapes=[
            pltpu.VMEM((B, WIN, HD), BF),
            pltpu.VMEM((B, WIN, HD), BF),
            pltpu.VMEM((B, WIN, HD), BF),
            pltpu.VMEM((B, WIN, HD), BF),
            pltpu.SemaphoreType.DMA((4,)),
            pltpu.SemaphoreType.DMA((4,)),
        ],
        compiler_params=pltpu.CompilerParams(collective_id=0),
    )(xb, Wqb, K2, V2, Wob)
